# Initial kernel scaffold; baseline (speedup 1.0000x reference)
import jax
import jax.numpy as jnp
from jax.experimental import pallas as pl


def kernel(x, hyperedge_all, lin1_W, lin1_b, bn1_w, bn1_b, hconv_W, hconv_b, bn2_w, bn2_b, lin2_W, lin2_b, ln_w, ln_b):
    raise NotImplementedError("write your pallas kernel here")


# speedup vs baseline: 14.0524x; 14.0524x over previous
import functools

import jax
import jax.numpy as jnp
from jax import lax
from jax.experimental import pallas as pl
from jax.experimental.pallas import tpu as pltpu
from jax.experimental.pallas import tpu_sc as plsc

EPS = 1e-5
_S1 = 1.0 / (1.0 + EPS) ** 0.5

_NC = 2
_NS = 16
_NW = _NC * _NS


def _leaky(v):
    return jnp.where(v >= 0, v, 0.2 * v)


def _front_body(x_ref, w1t_ref, b1_ref, g1_ref, be1_ref, wct_ref,
                h_ref, xt_ref):
    h = jnp.dot(x_ref[...], w1t_ref[...], preferred_element_type=jnp.float32)
    h = _leaky(h + b1_ref[...])
    h = h * (_S1 * g1_ref[...]) + be1_ref[...]
    h_ref[...] = h
    xt = jnp.dot(h, wct_ref[...], preferred_element_type=jnp.float32)
    xt_ref[:, :128] = xt
    r = xt.shape[0]
    lane = lax.broadcasted_iota(jnp.int32, (r, 16), 1)
    xt_ref[:, 128:144] = jnp.where(lane == 0, 1.0, 0.0)


def _tc_front(x2d, w1t, b1, g1, be1, wct, n, blk):
    grid = n // blk
    return pl.pallas_call(
        _front_body,
        grid=(grid,),
        in_specs=[
            pl.BlockSpec((blk, 128), lambda i: (i, 0)),
            pl.BlockSpec((128, 128), lambda i: (0, 0)),
            pl.BlockSpec((1, 128), lambda i: (0, 0)),
            pl.BlockSpec((1, 128), lambda i: (0, 0)),
            pl.BlockSpec((1, 128), lambda i: (0, 0)),
            pl.BlockSpec((128, 128), lambda i: (0, 0)),
        ],
        out_specs=[
            pl.BlockSpec((blk, 128), lambda i: (i, 0)),
            pl.BlockSpec((blk, 144), lambda i: (i, 0)),
        ],
        out_shape=[
            jax.ShapeDtypeStruct((n, 128), jnp.float32),
            jax.ShapeDtypeStruct((n, 144), jnp.float32),
        ],
    )(x2d, w1t, b1, g1, be1, wct)


def _sc_pass(table, gidx, sidx):
    n, w = table.shape
    nw, nch, k = gidx.shape
    per_sub = n // _NS
    full, rem = per_sub // k, per_sub % k

    @functools.partial(
        pl.kernel,
        out_type=jax.ShapeDtypeStruct((_NC, n, w), jnp.float32),
        mesh=plsc.VectorSubcoreMesh(core_axis_name="c", subcore_axis_name="s",
                                    num_cores=_NC, num_subcores=_NS),
        scratch_types=[
            pltpu.VMEM((nch, k), jnp.int32),
            pltpu.VMEM((nch, k), jnp.int32),
            pltpu.VMEM((2, k, w), jnp.float32),
            pltpu.VMEM_SHARED((n, w), jnp.float32),
            pltpu.SemaphoreType.DMA,
        ],
        compiler_params=pltpu.CompilerParams(use_tc_tiling_on_sc=False),
    )
    def body(table_hbm, gidx_hbm, sidx_hbm, out_hbm,
             gidx_v, sidx_v, rows_v, acc_sh, sem):
        cid = lax.axis_index("c")
        sid = lax.axis_index("s")
        wid = cid * _NS + sid

        zero16 = jnp.zeros((16,), jnp.float32)
        for i in range(k):
            for j in range(w // 16):
                rows_v[0, i, pl.ds(j * 16, 16)] = zero16
        base = sid * per_sub
        for t in range(full):
            pltpu.sync_copy(rows_v.at[0], acc_sh.at[pl.ds(base + t * k, k)])
        if rem:
            pltpu.sync_copy(rows_v.at[0, pl.ds(0, rem)],
                            acc_sh.at[pl.ds(base + full * k, rem)])
        plsc.subcore_barrier()

        pltpu.sync_copy(gidx_hbm.at[wid], gidx_v)
        pltpu.sync_copy(sidx_hbm.at[wid], sidx_v)

        def gather(c, buf):
            return pltpu.make_async_copy(
                table_hbm.at[gidx_v.at[c]], rows_v.at[buf], sem)

        gather(0, 0).start()

        def chunk(c, carry):
            buf = lax.rem(c, 2)

            @pl.when(c + 1 < nch)
            def _():
                gather(c + 1, 1 - buf).start()

            gather(c, buf).wait()
            pltpu.sync_copy(rows_v.at[buf], acc_sh.at[sidx_v.at[c]], add=True)
            return carry

        lax.fori_loop(0, nch, chunk, 0)
        plsc.subcore_barrier()

        pltpu.sync_copy(acc_sh.at[pl.ds(base, per_sub)],
                        out_hbm.at[cid, pl.ds(base, per_sub)])

    return body(table, gidx, sidx)


def _comb_body(p0_ref, p1_ref, o_ref):
    s = p0_ref[...] + p1_ref[...]
    cnt = s[:, 128:129]
    inv = jnp.where(cnt == 0, 0.0, 1.0 / jnp.where(cnt == 0, 1.0, cnt))
    o_ref[:, :128] = s[:, :128] * inv
    r = s.shape[0]
    lane = lax.broadcasted_iota(jnp.int32, (r, 16), 1)
    o_ref[:, 128:144] = jnp.where(lane == 0, 1.0, 0.0)


def _tc_comb(p0, p1, n, blk):
    return pl.pallas_call(
        _comb_body,
        grid=(n // blk,),
        in_specs=[
            pl.BlockSpec((blk, 144), lambda i: (i, 0)),
            pl.BlockSpec((blk, 144), lambda i: (i, 0)),
        ],
        out_specs=pl.BlockSpec((blk, 144), lambda i: (i, 0)),
        out_shape=jax.ShapeDtypeStruct((n, 144), jnp.float32),
    )(p0, p1)


def _back_body(src_ref, h_ref, q0_ref, q1_ref, bc_ref, g2_ref, be2_ref,
               w2t_ref, b2_ref, lw_ref, lb_ref, o_ref):
    q = q0_ref[...] + q1_ref[...]
    dv = q[:, 128:129]
    dinv = jnp.where(dv == 0, 0.0, 1.0 / jnp.where(dv == 0, 1.0, dv))
    hh = h_ref[...] + q[:, :128] * dinv + bc_ref[...]
    hh = hh * (_S1 * g2_ref[...]) + be2_ref[...]
    g = jnp.dot(hh, w2t_ref[...], preferred_element_type=jnp.float32)
    g = _leaky(g + b2_ref[...])
    o = src_ref[...] + g
    mu = jnp.mean(o, axis=1, keepdims=True)
    var = jnp.mean((o - mu) ** 2, axis=1, keepdims=True)
    o_ref[...] = (o - mu) / jnp.sqrt(var + EPS) * lw_ref[...] + lb_ref[...]


def _tc_back(src2d, h, q0, q1, bc, g2, be2, w2t, b2, lw, lb, n, blk):
    vec = pl.BlockSpec((1, 128), lambda i: (0, 0))
    return pl.pallas_call(
        _back_body,
        grid=(n // blk,),
        in_specs=[
            pl.BlockSpec((blk, 128), lambda i: (i, 0)),
            pl.BlockSpec((blk, 128), lambda i: (i, 0)),
            pl.BlockSpec((blk, 144), lambda i: (i, 0)),
            pl.BlockSpec((blk, 144), lambda i: (i, 0)),
            vec, vec, vec,
            pl.BlockSpec((128, 128), lambda i: (0, 0)),
            vec, vec, vec,
        ],
        out_specs=pl.BlockSpec((blk, 128), lambda i: (i, 0)),
        out_shape=jax.ShapeDtypeStruct((n, 128), jnp.float32),
    )(src2d, h, q0, q1, bc, g2, be2, w2t, b2, lw, lb)


def kernel(x, hyperedge_all, lin1_W, lin1_b, bn1_w, bn1_b, hconv_W, hconv_b,
           bn2_w, bn2_b, lin2_W, lin2_b, ln_w, ln_b):
    b_, n, c = x.shape
    nnz = hyperedge_all.shape[1]
    per_w = nnz // _NW
    k = 40
    nch = per_w // k

    x2d = x.reshape(n, b_ * c)
    row = hyperedge_all[0].reshape(_NW, nch, k)
    col = hyperedge_all[1].reshape(_NW, nch, k)

    blk = 1000
    h, xt_ext = _tc_front(
        x2d, lin1_W.T, lin1_b.reshape(1, -1), bn1_w.reshape(1, -1),
        bn1_b.reshape(1, -1), hconv_W.T, n, blk)

    p = _sc_pass(xt_ext, row, col)
    out_e = _tc_comb(p[0], p[1], n, blk)
    q = _sc_pass(out_e, col, row)

    out2d = _tc_back(
        x2d, h, q[0], q[1], hconv_b.reshape(1, -1), bn2_w.reshape(1, -1),
        bn2_b.reshape(1, -1), lin2_W.T, lin2_b.reshape(1, -1),
        ln_w.reshape(1, -1), ln_b.reshape(1, -1), n, blk)
    return out2d.reshape(b_, n, c)
